# Initial kernel scaffold; baseline (speedup 1.0000x reference)
import jax
import jax.numpy as jnp
from jax.experimental import pallas as pl


def kernel(x, gate_W, gate_b, fc1_W, fc1_b, fc2_W, fc2_b):
    raise NotImplementedError("write your pallas kernel here")


# speedup vs baseline: 1.6922x; 1.6922x over previous
import functools

import jax
import jax.numpy as jnp
from jax import lax
from jax.experimental import pallas as pl
from jax.experimental.pallas import tpu as pltpu
from jax.experimental.pallas import tpu_sc as plsc

_T = 2048
_C = 1024
_F = 4096
_E = 8
_EPAD = 128
_M = 128
_NB = _T * 2 // _M + _E
_SPAD = _NB * _M
_FB = 512
_NF = _F // _FB
_NC, _NS = 2, 16
_NW = _NC * _NS
_TPW = _T // _NW


def _gelu_exact(h):
    return 0.5 * h * (1.0 + lax.erf(h * 0.7071067811865476))


def _router_body(x_ref, gw_ref, gb_ref,
                 pos0_ref, pos1_ref, w0_ref, w1_ref, be_ref, used_ref):
    lane = lax.broadcasted_iota(jnp.int32, (_T, _EPAD), 1).astype(jnp.float32)
    valid_row = lax.broadcasted_iota(jnp.int32, (1, _EPAD), 1) < _E
    logits = jnp.dot(x_ref[...], gw_ref[...], preferred_element_type=jnp.float32)
    logits = logits + gb_ref[...]
    neg = -1e30
    l = jnp.where(lane < _E, logits, neg)
    m1 = jnp.max(l, axis=1, keepdims=True)
    i1 = jnp.min(jnp.where(l == m1, lane, 1e9), axis=1, keepdims=True)
    sel1 = lane == i1
    l2 = jnp.where(sel1, neg, l)
    m2 = jnp.max(l2, axis=1, keepdims=True)
    i2 = jnp.min(jnp.where(l2 == m2, lane, 1e9), axis=1, keepdims=True)
    sel2 = lane == i2
    e2 = jnp.exp(m2 - m1)
    w0 = 1.0 / (1.0 + e2)
    w1 = e2 / (1.0 + e2)
    oh = sel1.astype(jnp.float32) + sel2.astype(jnp.float32)
    acc = oh
    k = 1
    while k < _T:
        shifted = jnp.concatenate(
            [jnp.zeros((k, _EPAD), jnp.float32), acc[:_T - k, :]], axis=0)
        acc = acc + shifted
        k *= 2
    ranks = acc - oh
    counts = acc[_T - 1:_T, :]
    ub = jnp.floor((counts + (_M - 1)) * (1.0 / _M))
    pc = ub * _M
    tri = (lax.broadcasted_iota(jnp.int32, (_EPAD, _EPAD), 0) <
           lax.broadcasted_iota(jnp.int32, (_EPAD, _EPAD), 1)).astype(jnp.float32)
    poff = jnp.dot(pc, tri, preferred_element_type=jnp.float32)
    cb = (poff + pc) * (1.0 / _M)
    cb_m = jnp.where(valid_row, cb, 1e9)
    used_f = cb[0:1, _E - 1:_E]
    rowb = lax.broadcasted_iota(jnp.int32, (_NB, _EPAD), 0).astype(jnp.float32)
    be = jnp.sum((rowb >= cb_m).astype(jnp.float32), axis=1, keepdims=True)
    rowcol = lax.broadcasted_iota(jnp.int32, (_NB, 1), 0).astype(jnp.float32)
    be_last = jnp.sum(jnp.where(rowcol == used_f - 1.0, be, 0.0),
                      axis=0, keepdims=True)
    be_fin = jnp.where(rowcol < used_f, be, be_last)
    base = ranks + poff
    pos0 = jnp.sum(jnp.where(sel1, base, 0.0), axis=1, keepdims=True)
    pos1 = jnp.sum(jnp.where(sel2, base, 0.0), axis=1, keepdims=True)
    pos0_ref[...] = pos0.astype(jnp.int32)
    pos1_ref[...] = pos1.astype(jnp.int32)
    w0_ref[...] = jnp.broadcast_to(w0, (_T, 16))
    w1_ref[...] = jnp.broadcast_to(w1, (_T, 16))
    be_ref[...] = be_fin.astype(jnp.int32)
    used_ref[...] = used_f.astype(jnp.int32)


def _router_call(x2d, gwp, gbp, interpret=False):
    return pl.pallas_call(
        _router_body,
        out_shape=[
            jax.ShapeDtypeStruct((_T, 1), jnp.int32),
            jax.ShapeDtypeStruct((_T, 1), jnp.int32),
            jax.ShapeDtypeStruct((_T, 16), jnp.float32),
            jax.ShapeDtypeStruct((_T, 16), jnp.float32),
            jax.ShapeDtypeStruct((_NB, 1), jnp.int32),
            jax.ShapeDtypeStruct((1, 1), jnp.int32),
        ],
        interpret=interpret,
    )(x2d, gwp, gbp)


def _gmm_body(be_s, used_s, xg_ref, w1_ref, b1_ref, w2_ref, b2_ref, out_ref):
    b = pl.program_id(0)
    f = pl.program_id(1)

    @pl.when(f == 0)
    def _init():
        out_ref[...] = jnp.broadcast_to(b2_ref[0], (_M, _C))

    @pl.when(b < used_s[0])
    def _compute():
        h = jnp.dot(xg_ref[...], w1_ref[0], preferred_element_type=jnp.float32)
        h = _gelu_exact(h + b1_ref[0])
        out_ref[...] += jnp.dot(h, w2_ref[0], preferred_element_type=jnp.float32)


def _gmm_call(be, used, xg, fc1_W, fc1_b, fc2_W, fc2_b, interpret=False):
    grid_spec = pltpu.PrefetchScalarGridSpec(
        num_scalar_prefetch=2,
        grid=(_NB, _NF),
        in_specs=[
            pl.BlockSpec(
                (_M, _C),
                lambda b, f, be, used: (jnp.where(b < used[0], b, used[0] - 1), 0)),
            pl.BlockSpec(
                (1, _C, _FB),
                lambda b, f, be, used: (be[b], 0, jnp.where(b < used[0], f, _NF - 1))),
            pl.BlockSpec(
                (1, 1, _FB),
                lambda b, f, be, used: (be[b], 0, jnp.where(b < used[0], f, _NF - 1))),
            pl.BlockSpec(
                (1, _FB, _C),
                lambda b, f, be, used: (be[b], jnp.where(b < used[0], f, _NF - 1), 0)),
            pl.BlockSpec((1, 1, _C), lambda b, f, be, used: (be[b], 0, 0)),
        ],
        out_specs=pl.BlockSpec((_M, _C), lambda b, f, be, used: (b, 0)),
    )
    return pl.pallas_call(
        _gmm_body,
        grid_spec=grid_spec,
        out_shape=jax.ShapeDtypeStruct((_SPAD, _C), jnp.float32),
        compiler_params=pltpu.CompilerParams(
            dimension_semantics=("arbitrary", "arbitrary")),
        interpret=interpret,
    )(be, used, xg, fc1_W, fc1_b.reshape(_E, 1, _F), fc2_W,
      fc2_b.reshape(_E, 1, _C))


def _dispatch_call(x2d, pos0, pos1):
    mesh = plsc.VectorSubcoreMesh(core_axis_name="c", subcore_axis_name="s")

    @functools.partial(
        pl.kernel, mesh=mesh,
        out_type=jax.ShapeDtypeStruct((_SPAD, _C), jnp.float32),
        scratch_types=[
            pltpu.VMEM((_TPW,), jnp.int32),
            pltpu.VMEM((_TPW,), jnp.int32),
            pltpu.VMEM((_TPW, _C), jnp.float32),
            pltpu.SemaphoreType.DMA,
        ],
    )
    def dispatch(x_hbm, pos0_hbm, pos1_hbm, xg_hbm, idx0_v, idx1_v, rows_v, sem):
        wid = lax.axis_index("s") * _NC + lax.axis_index("c")
        row0 = wid * _TPW
        pltpu.sync_copy(x_hbm.at[pl.ds(row0, _TPW)], rows_v)
        pltpu.sync_copy(pos0_hbm.at[pl.ds(row0, _TPW)], idx0_v)
        pltpu.sync_copy(pos1_hbm.at[pl.ds(row0, _TPW)], idx1_v)
        c0 = pltpu.async_copy(rows_v, xg_hbm.at[idx0_v], sem)
        c1 = pltpu.async_copy(rows_v, xg_hbm.at[idx1_v], sem)
        c0.wait()
        c1.wait()

    return dispatch(x2d, pos0, pos1)


def _combine_call(yg, pos0, pos1, w0e, w1e):
    mesh = plsc.VectorSubcoreMesh(core_axis_name="c", subcore_axis_name="s")
    ch_n = 32

    @functools.partial(
        pl.kernel, mesh=mesh,
        out_type=jax.ShapeDtypeStruct((_T, _C), jnp.float32),
        scratch_types=[
            pltpu.VMEM((ch_n,), jnp.int32),
            pltpu.VMEM((ch_n,), jnp.int32),
            pltpu.VMEM((ch_n, _C), jnp.float32),
            pltpu.VMEM((ch_n, _C), jnp.float32),
            pltpu.VMEM((ch_n, 16), jnp.float32),
            pltpu.VMEM((ch_n, 16), jnp.float32),
            pltpu.SemaphoreType.DMA,
        ],
    )
    def combine(yg_hbm, pos0_hbm, pos1_hbm, w0_hbm, w1_hbm, out_hbm,
                idx0_v, idx1_v, r0_v, r1_v, w0_v, w1_v, sem):
        wid = lax.axis_index("s") * _NC + lax.axis_index("c")
        for ch in range(_TPW // ch_n):
            base = wid * _TPW + ch * ch_n
            pltpu.sync_copy(pos0_hbm.at[pl.ds(base, ch_n)], idx0_v)
            pltpu.sync_copy(pos1_hbm.at[pl.ds(base, ch_n)], idx1_v)
            pltpu.sync_copy(w0_hbm.at[pl.ds(base, ch_n)], w0_v)
            pltpu.sync_copy(w1_hbm.at[pl.ds(base, ch_n)], w1_v)
            g0 = pltpu.async_copy(yg_hbm.at[idx0_v], r0_v, sem)
            g1 = pltpu.async_copy(yg_hbm.at[idx1_v], r1_v, sem)
            g0.wait()
            g1.wait()

            def body(i, carry):
                a = w0_v[i, :]
                b = w1_v[i, :]
                for j in range(_C // 16):
                    sl = pl.ds(j * 16, 16)
                    r0_v[i, sl] = r0_v[i, sl] * a + r1_v[i, sl] * b
                return carry

            lax.fori_loop(0, ch_n, body, 0)
            pltpu.sync_copy(r0_v, out_hbm.at[pl.ds(base, ch_n)])

    return combine(yg, pos0, pos1, w0e, w1e)


def kernel(x, gate_W, gate_b, fc1_W, fc1_b, fc2_W, fc2_b):
    B, T, C = x.shape
    x2d = x.reshape(T, C)
    gwp = jnp.pad(gate_W, ((0, 0), (0, _EPAD - _E)))
    gbp = jnp.pad(gate_b, (0, _EPAD - _E)).reshape(1, _EPAD)
    pos0, pos1, w0e, w1e, be, used = _router_call(x2d, gwp, gbp)
    pos0f = pos0.reshape(_T)
    pos1f = pos1.reshape(_T)
    xg = _dispatch_call(x2d, pos0f, pos1f)
    yg = _gmm_call(be.reshape(_NB), used.reshape(1), xg,
                   fc1_W, fc1_b, fc2_W, fc2_b)
    out = _combine_call(yg, pos0f, pos1f, w0e, w1e)
    return out.reshape(B, T, C)


# speedup vs baseline: 1.9534x; 1.1544x over previous
import functools

import jax
import jax.numpy as jnp
from jax import lax
from jax.experimental import pallas as pl
from jax.experimental.pallas import tpu as pltpu
from jax.experimental.pallas import tpu_sc as plsc

_T = 2048
_C = 1024
_F = 4096
_E = 8
_EPAD = 128
_M = 128
_NB = _T * 2 // _M + _E
_SPAD = _NB * _M
_FB = 512
_NF = _F // _FB
_NC, _NS = 2, 16
_NW = _NC * _NS
_TPW = _T // _NW


def _gelu_exact(h):
    return 0.5 * h * (1.0 + lax.erf(h * 0.7071067811865476))


def _router_body(x_ref, gw_ref, gb_ref,
                 pos0_ref, pos1_ref, w0_ref, w1_ref, be_ref, used_ref):
    lane = lax.broadcasted_iota(jnp.int32, (_T, _EPAD), 1).astype(jnp.float32)
    valid_row = lax.broadcasted_iota(jnp.int32, (1, _EPAD), 1) < _E
    logits = jnp.dot(x_ref[...], gw_ref[...], preferred_element_type=jnp.float32)
    logits = logits + gb_ref[...]
    neg = -1e30
    l = jnp.where(lane < _E, logits, neg)
    m1 = jnp.max(l, axis=1, keepdims=True)
    i1 = jnp.min(jnp.where(l == m1, lane, 1e9), axis=1, keepdims=True)
    sel1 = lane == i1
    l2 = jnp.where(sel1, neg, l)
    m2 = jnp.max(l2, axis=1, keepdims=True)
    i2 = jnp.min(jnp.where(l2 == m2, lane, 1e9), axis=1, keepdims=True)
    sel2 = lane == i2
    e2 = jnp.exp(m2 - m1)
    w0 = 1.0 / (1.0 + e2)
    w1 = e2 / (1.0 + e2)
    oh = sel1.astype(jnp.float32) + sel2.astype(jnp.float32)
    acc = oh
    k = 1
    while k < _T:
        shifted = jnp.concatenate(
            [jnp.zeros((k, _EPAD), jnp.float32), acc[:_T - k, :]], axis=0)
        acc = acc + shifted
        k *= 2
    ranks = acc - oh
    counts = acc[_T - 1:_T, :]
    ub = jnp.floor((counts + (_M - 1)) * (1.0 / _M))
    pc = ub * _M
    tri = (lax.broadcasted_iota(jnp.int32, (_EPAD, _EPAD), 0) <
           lax.broadcasted_iota(jnp.int32, (_EPAD, _EPAD), 1)).astype(jnp.float32)
    poff = jnp.dot(pc, tri, preferred_element_type=jnp.float32)
    cb = (poff + pc) * (1.0 / _M)
    cb_m = jnp.where(valid_row, cb, 1e9)
    used_f = cb[0:1, _E - 1:_E]
    rowb = lax.broadcasted_iota(jnp.int32, (_NB, _EPAD), 0).astype(jnp.float32)
    be = jnp.sum((rowb >= cb_m).astype(jnp.float32), axis=1, keepdims=True)
    rowcol = lax.broadcasted_iota(jnp.int32, (_NB, 1), 0).astype(jnp.float32)
    be_last = jnp.sum(jnp.where(rowcol == used_f - 1.0, be, 0.0),
                      axis=0, keepdims=True)
    be_fin = jnp.where(rowcol < used_f, be, be_last)
    base = ranks + poff
    pos0 = jnp.sum(jnp.where(sel1, base, 0.0), axis=1, keepdims=True)
    pos1 = jnp.sum(jnp.where(sel2, base, 0.0), axis=1, keepdims=True)
    pos0_ref[...] = pos0.astype(jnp.int32)
    pos1_ref[...] = pos1.astype(jnp.int32)
    w0_ref[...] = jnp.broadcast_to(w0, (_T, 16))
    w1_ref[...] = jnp.broadcast_to(w1, (_T, 16))
    be_ref[...] = be_fin.astype(jnp.int32)
    used_ref[...] = used_f.astype(jnp.int32)


def _router_call(x2d, gwp, gbp, interpret=False):
    return pl.pallas_call(
        _router_body,
        out_shape=[
            jax.ShapeDtypeStruct((_T, 1), jnp.int32),
            jax.ShapeDtypeStruct((_T, 1), jnp.int32),
            jax.ShapeDtypeStruct((_T, 16), jnp.float32),
            jax.ShapeDtypeStruct((_T, 16), jnp.float32),
            jax.ShapeDtypeStruct((_NB, 1), jnp.int32),
            jax.ShapeDtypeStruct((1, 1), jnp.int32),
        ],
        interpret=interpret,
    )(x2d, gwp, gbp)


def _gmm_body(be_s, used_s, xg_ref, w1_ref, b1_ref, w2_ref, b2_ref, out_hbm,
              xgb, w1c, w2c, acc, sem):
    f = pl.program_id(0)
    b = pl.program_id(1)
    valid = b < used_s[0]
    off = pl.multiple_of(b * _M, _M)

    @pl.when((f == 0) & valid)
    def _stage_x():
        xgb[pl.ds(off, _M), :] = xg_ref[...].astype(jnp.bfloat16)

    e_prev = be_s[jnp.maximum(b - 1, 0)]
    changed = (b == 0) | (be_s[b] != e_prev)

    @pl.when(changed)
    def _cast_w():
        w1c[...] = w1_ref[0].astype(jnp.bfloat16)
        w2c[...] = w2_ref[0].astype(jnp.bfloat16)

    @pl.when(valid)
    def _compute():
        xb = xgb[pl.ds(off, _M), :]
        h = jnp.dot(xb, w1c[...], preferred_element_type=jnp.float32)
        h = _gelu_exact(h + b1_ref[0])
        upd = jnp.dot(h.astype(jnp.bfloat16), w2c[...],
                      preferred_element_type=jnp.float32)

        @pl.when(f == 0)
        def _():
            acc[pl.ds(off, _M), :] = jnp.broadcast_to(b2_ref[0], (_M, _C)) + upd

        @pl.when(f > 0)
        def _():
            acc[pl.ds(off, _M), :] += upd

        @pl.when(f == _NF - 1)
        def _flush():
            cp = pltpu.make_async_copy(
                acc.at[pl.ds(off, _M)], out_hbm.at[pl.ds(off, _M)], sem)
            cp.start()
            cp.wait()


def _gmm_call(be, used, xg, fc1_W, fc1_b, fc2_W, fc2_b, interpret=False):
    grid_spec = pltpu.PrefetchScalarGridSpec(
        num_scalar_prefetch=2,
        grid=(_NF, _NB),
        in_specs=[
            pl.BlockSpec(
                (_M, _C),
                lambda f, b, be, used: (
                    jnp.where(f == 0,
                              jnp.where(b < used[0], b, used[0] - 1), 0), 0)),
            pl.BlockSpec(
                (1, _C, _FB), lambda f, b, be, used: (be[b], 0, f)),
            pl.BlockSpec(
                (1, 1, _FB), lambda f, b, be, used: (be[b], 0, f)),
            pl.BlockSpec(
                (1, _FB, _C), lambda f, b, be, used: (be[b], f, 0)),
            pl.BlockSpec((1, 1, _C), lambda f, b, be, used: (be[b], 0, 0)),
        ],
        out_specs=pl.BlockSpec(memory_space=pltpu.MemorySpace.HBM),
        scratch_shapes=[
            pltpu.VMEM((_SPAD, _C), jnp.bfloat16),
            pltpu.VMEM((_C, _FB), jnp.bfloat16),
            pltpu.VMEM((_FB, _C), jnp.bfloat16),
            pltpu.VMEM((_SPAD, _C), jnp.float32),
            pltpu.SemaphoreType.DMA,
        ],
    )
    return pl.pallas_call(
        _gmm_body,
        grid_spec=grid_spec,
        out_shape=jax.ShapeDtypeStruct((_SPAD, _C), jnp.float32),
        compiler_params=pltpu.CompilerParams(
            dimension_semantics=("arbitrary", "arbitrary")),
        interpret=interpret,
    )(be, used, xg, fc1_W, fc1_b.reshape(_E, 1, _F), fc2_W,
      fc2_b.reshape(_E, 1, _C))


def _dispatch_call(x2d, pos0, pos1):
    mesh = plsc.VectorSubcoreMesh(core_axis_name="c", subcore_axis_name="s")

    @functools.partial(
        pl.kernel, mesh=mesh,
        out_type=jax.ShapeDtypeStruct((_SPAD, _C), jnp.float32),
        scratch_types=[
            pltpu.VMEM((_TPW,), jnp.int32),
            pltpu.VMEM((_TPW,), jnp.int32),
            pltpu.VMEM((_TPW, _C), jnp.float32),
            pltpu.SemaphoreType.DMA,
        ],
    )
    def dispatch(x_hbm, pos0_hbm, pos1_hbm, xg_hbm, idx0_v, idx1_v, rows_v, sem):
        wid = lax.axis_index("s") * _NC + lax.axis_index("c")
        row0 = wid * _TPW
        pltpu.sync_copy(x_hbm.at[pl.ds(row0, _TPW)], rows_v)
        pltpu.sync_copy(pos0_hbm.at[pl.ds(row0, _TPW)], idx0_v)
        pltpu.sync_copy(pos1_hbm.at[pl.ds(row0, _TPW)], idx1_v)
        c0 = pltpu.async_copy(rows_v, xg_hbm.at[idx0_v], sem)
        c1 = pltpu.async_copy(rows_v, xg_hbm.at[idx1_v], sem)
        c0.wait()
        c1.wait()

    return dispatch(x2d, pos0, pos1)


def _combine_call(yg, pos0, pos1, w0e, w1e):
    mesh = plsc.VectorSubcoreMesh(core_axis_name="c", subcore_axis_name="s")
    ch_n = 32

    @functools.partial(
        pl.kernel, mesh=mesh,
        out_type=jax.ShapeDtypeStruct((_T, _C), jnp.float32),
        scratch_types=[
            pltpu.VMEM((ch_n,), jnp.int32),
            pltpu.VMEM((ch_n,), jnp.int32),
            pltpu.VMEM((ch_n, _C), jnp.float32),
            pltpu.VMEM((ch_n, _C), jnp.float32),
            pltpu.VMEM((ch_n, 16), jnp.float32),
            pltpu.VMEM((ch_n, 16), jnp.float32),
            pltpu.SemaphoreType.DMA,
        ],
    )
    def combine(yg_hbm, pos0_hbm, pos1_hbm, w0_hbm, w1_hbm, out_hbm,
                idx0_v, idx1_v, r0_v, r1_v, w0_v, w1_v, sem):
        wid = lax.axis_index("s") * _NC + lax.axis_index("c")
        for ch in range(_TPW // ch_n):
            base = wid * _TPW + ch * ch_n
            pltpu.sync_copy(pos0_hbm.at[pl.ds(base, ch_n)], idx0_v)
            pltpu.sync_copy(pos1_hbm.at[pl.ds(base, ch_n)], idx1_v)
            pltpu.sync_copy(w0_hbm.at[pl.ds(base, ch_n)], w0_v)
            pltpu.sync_copy(w1_hbm.at[pl.ds(base, ch_n)], w1_v)
            g0 = pltpu.async_copy(yg_hbm.at[idx0_v], r0_v, sem)
            g1 = pltpu.async_copy(yg_hbm.at[idx1_v], r1_v, sem)
            g0.wait()
            g1.wait()

            def body(i, carry):
                a = w0_v[i, :]
                b = w1_v[i, :]
                for j in range(_C // 16):
                    sl = pl.ds(j * 16, 16)
                    r0_v[i, sl] = r0_v[i, sl] * a + r1_v[i, sl] * b
                return carry

            lax.fori_loop(0, ch_n, body, 0)
            pltpu.sync_copy(r0_v, out_hbm.at[pl.ds(base, ch_n)])

    return combine(yg, pos0, pos1, w0e, w1e)


def kernel(x, gate_W, gate_b, fc1_W, fc1_b, fc2_W, fc2_b):
    B, T, C = x.shape
    x2d = x.reshape(T, C)
    gwp = jnp.pad(gate_W, ((0, 0), (0, _EPAD - _E)))
    gbp = jnp.pad(gate_b, (0, _EPAD - _E)).reshape(1, _EPAD)
    pos0, pos1, w0e, w1e, be, used = _router_call(x2d, gwp, gbp)
    pos0f = pos0.reshape(_T)
    pos1f = pos1.reshape(_T)
    xg = _dispatch_call(x2d, pos0f, pos1f)
    yg = _gmm_call(be.reshape(_NB), used.reshape(1), xg,
                   fc1_W, fc1_b, fc2_W, fc2_b)
    out = _combine_call(yg, pos0f, pos1f, w0e, w1e)
    return out.reshape(B, T, C)


# speedup vs baseline: 2.5683x; 1.3147x over previous
import functools

import jax
import jax.numpy as jnp
from jax import lax
from jax.experimental import pallas as pl
from jax.experimental.pallas import tpu as pltpu
from jax.experimental.pallas import tpu_sc as plsc

_T = 2048
_C = 1024
_F = 4096
_E = 8
_EPAD = 128
_M = 128
_NB = _T * 2 // _M + _E
_SPAD = _NB * _M
_FB = 1024
_NF = _F // _FB
_NC, _NS = 2, 16
_NW = _NC * _NS
_TPW = _T // _NW


def _gelu_exact(h):
    return 0.5 * h * (1.0 + lax.erf(h * 0.7071067811865476))


def _router_body(x_ref, gw_ref, gb_ref,
                 pos0_ref, pos1_ref, w0_ref, w1_ref, be_ref, used_ref):
    lane = lax.broadcasted_iota(jnp.int32, (_T, _EPAD), 1).astype(jnp.float32)
    valid_row = lax.broadcasted_iota(jnp.int32, (1, _EPAD), 1) < _E
    logits = jnp.dot(x_ref[...], gw_ref[...], preferred_element_type=jnp.float32)
    logits = logits + gb_ref[...]
    neg = -1e30
    l = jnp.where(lane < _E, logits, neg)
    m1 = jnp.max(l, axis=1, keepdims=True)
    i1 = jnp.min(jnp.where(l == m1, lane, 1e9), axis=1, keepdims=True)
    sel1 = lane == i1
    l2 = jnp.where(sel1, neg, l)
    m2 = jnp.max(l2, axis=1, keepdims=True)
    i2 = jnp.min(jnp.where(l2 == m2, lane, 1e9), axis=1, keepdims=True)
    sel2 = lane == i2
    e2 = jnp.exp(m2 - m1)
    w0 = 1.0 / (1.0 + e2)
    w1 = e2 / (1.0 + e2)
    oh = sel1.astype(jnp.float32) + sel2.astype(jnp.float32)
    acc = oh
    k = 1
    while k < _T:
        shifted = jnp.concatenate(
            [jnp.zeros((k, _EPAD), jnp.float32), acc[:_T - k, :]], axis=0)
        acc = acc + shifted
        k *= 2
    ranks = acc - oh
    counts = acc[_T - 1:_T, :]
    ub = jnp.floor((counts + (_M - 1)) * (1.0 / _M))
    pc = ub * _M
    tri = (lax.broadcasted_iota(jnp.int32, (_EPAD, _EPAD), 0) <
           lax.broadcasted_iota(jnp.int32, (_EPAD, _EPAD), 1)).astype(jnp.float32)
    poff = jnp.dot(pc, tri, preferred_element_type=jnp.float32)
    cb = (poff + pc) * (1.0 / _M)
    cb_m = jnp.where(valid_row, cb, 1e9)
    used_f = cb[0:1, _E - 1:_E]
    rowb = lax.broadcasted_iota(jnp.int32, (_NB, _EPAD), 0).astype(jnp.float32)
    be = jnp.sum((rowb >= cb_m).astype(jnp.float32), axis=1, keepdims=True)
    rowcol = lax.broadcasted_iota(jnp.int32, (_NB, 1), 0).astype(jnp.float32)
    be_last = jnp.sum(jnp.where(rowcol == used_f - 1.0, be, 0.0),
                      axis=0, keepdims=True)
    be_fin = jnp.where(rowcol < used_f, be, be_last)
    base = ranks + poff
    pos0 = jnp.sum(jnp.where(sel1, base, 0.0), axis=1, keepdims=True)
    pos1 = jnp.sum(jnp.where(sel2, base, 0.0), axis=1, keepdims=True)
    pos0_ref[...] = pos0.astype(jnp.int32)
    pos1_ref[...] = pos1.astype(jnp.int32)
    w0_ref[...] = jnp.broadcast_to(w0, (_T, 16))
    w1_ref[...] = jnp.broadcast_to(w1, (_T, 16))
    be_ref[...] = be_fin.astype(jnp.int32)
    used_ref[...] = used_f.astype(jnp.int32)


def _router_call(x2d, gwp, gbp, interpret=False):
    return pl.pallas_call(
        _router_body,
        out_shape=[
            jax.ShapeDtypeStruct((_T, 1), jnp.int32),
            jax.ShapeDtypeStruct((_T, 1), jnp.int32),
            jax.ShapeDtypeStruct((_T, 16), jnp.float32),
            jax.ShapeDtypeStruct((_T, 16), jnp.float32),
            jax.ShapeDtypeStruct((_NB, 1), jnp.int32),
            jax.ShapeDtypeStruct((1, 1), jnp.int32),
        ],
        interpret=interpret,
    )(x2d, gwp, gbp)


def _gmm_body(be_s, used_s, xg_ref, w1_ref, b1_ref, w2_ref, b2_ref, out_hbm,
              xgb, w1c, w2c, acc, sem):
    f = pl.program_id(0)
    b = pl.program_id(1)
    valid = b < used_s[0]
    off = pl.multiple_of(b * _M, _M)

    @pl.when((f == 0) & valid)
    def _stage_x():
        xgb[pl.ds(off, _M), :] = xg_ref[...].astype(jnp.bfloat16)

    e_prev = be_s[jnp.maximum(b - 1, 0)]
    changed = (b == 0) | (be_s[b] != e_prev)

    @pl.when(changed)
    def _cast_w():
        w1c[...] = w1_ref[0].astype(jnp.bfloat16)
        w2c[...] = w2_ref[0].astype(jnp.bfloat16)

    @pl.when(valid)
    def _compute():
        xb = xgb[pl.ds(off, _M), :]
        h = jnp.dot(xb, w1c[...], preferred_element_type=jnp.float32)
        h = _gelu_exact(h + b1_ref[0])
        upd = jnp.dot(h.astype(jnp.bfloat16), w2c[...],
                      preferred_element_type=jnp.float32)

        @pl.when(f == 0)
        def _():
            acc[pl.ds(off, _M), :] = jnp.broadcast_to(b2_ref[0], (_M, _C)) + upd

        @pl.when(f > 0)
        def _():
            acc[pl.ds(off, _M), :] += upd

        @pl.when(f == _NF - 1)
        def _flush():
            cp = pltpu.make_async_copy(
                acc.at[pl.ds(off, _M)], out_hbm.at[pl.ds(off, _M)], sem)
            cp.start()
            cp.wait()


def _gmm_call(be, used, xg, fc1_W, fc1_b, fc2_W, fc2_b, interpret=False):
    grid_spec = pltpu.PrefetchScalarGridSpec(
        num_scalar_prefetch=2,
        grid=(_NF, _NB),
        in_specs=[
            pl.BlockSpec(
                (_M, _C),
                lambda f, b, be, used: (
                    jnp.where(f == 0,
                              jnp.where(b < used[0], b, used[0] - 1), 0), 0)),
            pl.BlockSpec(
                (1, _C, _FB), lambda f, b, be, used: (be[b], 0, f)),
            pl.BlockSpec(
                (1, 1, _FB), lambda f, b, be, used: (be[b], 0, f)),
            pl.BlockSpec(
                (1, _FB, _C), lambda f, b, be, used: (be[b], f, 0)),
            pl.BlockSpec((1, 1, _C), lambda f, b, be, used: (be[b], 0, 0)),
        ],
        out_specs=pl.BlockSpec(memory_space=pltpu.MemorySpace.HBM),
        scratch_shapes=[
            pltpu.VMEM((_SPAD, _C), jnp.bfloat16),
            pltpu.VMEM((_C, _FB), jnp.bfloat16),
            pltpu.VMEM((_FB, _C), jnp.bfloat16),
            pltpu.VMEM((_SPAD, _C), jnp.float32),
            pltpu.SemaphoreType.DMA,
        ],
    )
    return pl.pallas_call(
        _gmm_body,
        grid_spec=grid_spec,
        out_shape=jax.ShapeDtypeStruct((_SPAD, _C), jnp.float32),
        compiler_params=pltpu.CompilerParams(
            dimension_semantics=("arbitrary", "arbitrary")),
        interpret=interpret,
    )(be, used, xg, fc1_W, fc1_b.reshape(_E, 1, _F), fc2_W,
      fc2_b.reshape(_E, 1, _C))


def _dispatch_call(x2d, pos0, pos1):
    mesh = plsc.VectorSubcoreMesh(core_axis_name="c", subcore_axis_name="s")

    @functools.partial(
        pl.kernel, mesh=mesh,
        out_type=jax.ShapeDtypeStruct((_SPAD, _C), jnp.float32),
        scratch_types=[
            pltpu.VMEM((_TPW,), jnp.int32),
            pltpu.VMEM((_TPW,), jnp.int32),
            pltpu.VMEM((_TPW, _C), jnp.float32),
            pltpu.SemaphoreType.DMA,
        ],
    )
    def dispatch(x_hbm, pos0_hbm, pos1_hbm, xg_hbm, idx0_v, idx1_v, rows_v, sem):
        wid = lax.axis_index("s") * _NC + lax.axis_index("c")
        row0 = wid * _TPW
        pltpu.sync_copy(x_hbm.at[pl.ds(row0, _TPW)], rows_v)
        pltpu.sync_copy(pos0_hbm.at[pl.ds(row0, _TPW)], idx0_v)
        pltpu.sync_copy(pos1_hbm.at[pl.ds(row0, _TPW)], idx1_v)
        c0 = pltpu.async_copy(rows_v, xg_hbm.at[idx0_v], sem)
        c1 = pltpu.async_copy(rows_v, xg_hbm.at[idx1_v], sem)
        c0.wait()
        c1.wait()

    return dispatch(x2d, pos0, pos1)


def _combine_call(yg, pos0, pos1, w0e, w1e):
    mesh = plsc.VectorSubcoreMesh(core_axis_name="c", subcore_axis_name="s")
    ch_n = 32

    @functools.partial(
        pl.kernel, mesh=mesh,
        out_type=jax.ShapeDtypeStruct((_T, _C), jnp.float32),
        scratch_types=[
            pltpu.VMEM((ch_n,), jnp.int32),
            pltpu.VMEM((ch_n,), jnp.int32),
            pltpu.VMEM((ch_n, _C), jnp.float32),
            pltpu.VMEM((ch_n, _C), jnp.float32),
            pltpu.VMEM((ch_n, 16), jnp.float32),
            pltpu.VMEM((ch_n, 16), jnp.float32),
            pltpu.SemaphoreType.DMA,
        ],
    )
    def combine(yg_hbm, pos0_hbm, pos1_hbm, w0_hbm, w1_hbm, out_hbm,
                idx0_v, idx1_v, r0_v, r1_v, w0_v, w1_v, sem):
        wid = lax.axis_index("s") * _NC + lax.axis_index("c")
        for ch in range(_TPW // ch_n):
            base = wid * _TPW + ch * ch_n
            pltpu.sync_copy(pos0_hbm.at[pl.ds(base, ch_n)], idx0_v)
            pltpu.sync_copy(pos1_hbm.at[pl.ds(base, ch_n)], idx1_v)
            pltpu.sync_copy(w0_hbm.at[pl.ds(base, ch_n)], w0_v)
            pltpu.sync_copy(w1_hbm.at[pl.ds(base, ch_n)], w1_v)
            g0 = pltpu.async_copy(yg_hbm.at[idx0_v], r0_v, sem)
            g1 = pltpu.async_copy(yg_hbm.at[idx1_v], r1_v, sem)
            g0.wait()
            g1.wait()

            def body(i, carry):
                a = w0_v[i, :]
                b = w1_v[i, :]
                for j in range(_C // 16):
                    sl = pl.ds(j * 16, 16)
                    r0_v[i, sl] = r0_v[i, sl] * a + r1_v[i, sl] * b
                return carry

            lax.fori_loop(0, ch_n, body, 0)
            pltpu.sync_copy(r0_v, out_hbm.at[pl.ds(base, ch_n)])

    return combine(yg, pos0, pos1, w0e, w1e)


def kernel(x, gate_W, gate_b, fc1_W, fc1_b, fc2_W, fc2_b):
    B, T, C = x.shape
    x2d = x.reshape(T, C)
    gwp = jnp.pad(gate_W, ((0, 0), (0, _EPAD - _E)))
    gbp = jnp.pad(gate_b, (0, _EPAD - _E)).reshape(1, _EPAD)
    pos0, pos1, w0e, w1e, be, used = _router_call(x2d, gwp, gbp)
    pos0f = pos0.reshape(_T)
    pos1f = pos1.reshape(_T)
    xg = _dispatch_call(x2d, pos0f, pos1f)
    yg = _gmm_call(be.reshape(_NB), used.reshape(1), xg,
                   fc1_W, fc1_b, fc2_W, fc2_b)
    out = _combine_call(yg, pos0f, pos1f, w0e, w1e)
    return out.reshape(B, T, C)


# speedup vs baseline: 2.8735x; 1.1189x over previous
import functools

import jax
import jax.numpy as jnp
from jax import lax
from jax.experimental import pallas as pl
from jax.experimental.pallas import tpu as pltpu
from jax.experimental.pallas import tpu_sc as plsc

_T = 2048
_C = 1024
_F = 4096
_E = 8
_EPAD = 128
_M = 256
_NB = _T * 2 // _M + _E
_SPAD = _NB * _M
_FB = 1024
_NF = _F // _FB
_NC, _NS = 2, 16
_NW = _NC * _NS
_TPW = _T // _NW


def _gelu_exact(h):
    return 0.5 * h * (1.0 + lax.erf(h * 0.7071067811865476))


def _router_body(x_ref, gw_ref, gb_ref,
                 pos0_ref, pos1_ref, w0_ref, w1_ref, be_ref, used_ref):
    lane = lax.broadcasted_iota(jnp.int32, (_T, _EPAD), 1).astype(jnp.float32)
    valid_row = lax.broadcasted_iota(jnp.int32, (1, _EPAD), 1) < _E
    logits = jnp.dot(x_ref[...], gw_ref[...], preferred_element_type=jnp.float32)
    logits = logits + gb_ref[...]
    neg = -1e30
    l = jnp.where(lane < _E, logits, neg)
    m1 = jnp.max(l, axis=1, keepdims=True)
    i1 = jnp.min(jnp.where(l == m1, lane, 1e9), axis=1, keepdims=True)
    sel1 = lane == i1
    l2 = jnp.where(sel1, neg, l)
    m2 = jnp.max(l2, axis=1, keepdims=True)
    i2 = jnp.min(jnp.where(l2 == m2, lane, 1e9), axis=1, keepdims=True)
    sel2 = lane == i2
    e2 = jnp.exp(m2 - m1)
    w0 = 1.0 / (1.0 + e2)
    w1 = e2 / (1.0 + e2)
    oh = sel1.astype(jnp.float32) + sel2.astype(jnp.float32)
    acc = oh
    k = 1
    while k < _T:
        shifted = jnp.concatenate(
            [jnp.zeros((k, _EPAD), jnp.float32), acc[:_T - k, :]], axis=0)
        acc = acc + shifted
        k *= 2
    ranks = acc - oh
    counts = acc[_T - 1:_T, :]
    ub = jnp.floor((counts + (_M - 1)) * (1.0 / _M))
    pc = ub * _M
    tri = (lax.broadcasted_iota(jnp.int32, (_EPAD, _EPAD), 0) <
           lax.broadcasted_iota(jnp.int32, (_EPAD, _EPAD), 1)).astype(jnp.float32)
    poff = jnp.dot(pc, tri, preferred_element_type=jnp.float32)
    cb = (poff + pc) * (1.0 / _M)
    cb_m = jnp.where(valid_row, cb, 1e9)
    used_f = cb[0:1, _E - 1:_E]
    rowb = lax.broadcasted_iota(jnp.int32, (_NB, _EPAD), 0).astype(jnp.float32)
    be = jnp.sum((rowb >= cb_m).astype(jnp.float32), axis=1, keepdims=True)
    rowcol = lax.broadcasted_iota(jnp.int32, (_NB, 1), 0).astype(jnp.float32)
    be_last = jnp.sum(jnp.where(rowcol == used_f - 1.0, be, 0.0),
                      axis=0, keepdims=True)
    be_fin = jnp.where(rowcol < used_f, be, be_last)
    base = ranks + poff
    pos0 = jnp.sum(jnp.where(sel1, base, 0.0), axis=1, keepdims=True)
    pos1 = jnp.sum(jnp.where(sel2, base, 0.0), axis=1, keepdims=True)
    pos0_ref[...] = pos0.astype(jnp.int32)
    pos1_ref[...] = pos1.astype(jnp.int32)
    w0_ref[...] = jnp.broadcast_to(w0, (_T, 16))
    w1_ref[...] = jnp.broadcast_to(w1, (_T, 16))
    be_ref[...] = be_fin.astype(jnp.int32)
    used_ref[...] = used_f.astype(jnp.int32)


def _router_call(x2d, gwp, gbp, interpret=False):
    return pl.pallas_call(
        _router_body,
        out_shape=[
            jax.ShapeDtypeStruct((_T, 1), jnp.int32),
            jax.ShapeDtypeStruct((_T, 1), jnp.int32),
            jax.ShapeDtypeStruct((_T, 16), jnp.float32),
            jax.ShapeDtypeStruct((_T, 16), jnp.float32),
            jax.ShapeDtypeStruct((_NB, 1), jnp.int32),
            jax.ShapeDtypeStruct((1, 1), jnp.int32),
        ],
        interpret=interpret,
    )(x2d, gwp, gbp)


def _gmm_body(be_s, used_s, xg_ref, w1_ref, b1_ref, w2_ref, b2_ref, out_hbm,
              w1c, w2c, acc, sem):
    f = pl.program_id(0)
    b = pl.program_id(1)
    valid = b < used_s[0]
    off = pl.multiple_of(b * _M, _M)

    e_prev = be_s[jnp.maximum(b - 1, 0)]
    changed = (b == 0) | (be_s[b] != e_prev)

    @pl.when(changed)
    def _cast_w():
        w1c[...] = w1_ref[0].astype(jnp.bfloat16)
        w2c[...] = w2_ref[0].astype(jnp.bfloat16)

    @pl.when(valid)
    def _compute():
        xb = xg_ref[...].astype(jnp.bfloat16)
        h = jnp.dot(xb, w1c[...], preferred_element_type=jnp.float32)
        h = _gelu_exact(h + b1_ref[0])
        upd = jnp.dot(h.astype(jnp.bfloat16), w2c[...],
                      preferred_element_type=jnp.float32)

        @pl.when(f == 0)
        def _():
            acc[pl.ds(off, _M), :] = jnp.broadcast_to(b2_ref[0], (_M, _C)) + upd

        @pl.when(f > 0)
        def _():
            acc[pl.ds(off, _M), :] += upd

        @pl.when(f == _NF - 1)
        def _flush():
            cp = pltpu.make_async_copy(
                acc.at[pl.ds(off, _M)], out_hbm.at[pl.ds(off, _M)], sem)
            cp.start()
            cp.wait()


def _gmm_call(be, used, xg, fc1_W, fc1_b, fc2_W, fc2_b, interpret=False):
    grid_spec = pltpu.PrefetchScalarGridSpec(
        num_scalar_prefetch=2,
        grid=(_NF, _NB),
        in_specs=[
            pl.BlockSpec(
                (_M, _C),
                lambda f, b, be, used: (
                    jnp.where(b < used[0], b, used[0] - 1), 0)),
            pl.BlockSpec(
                (1, _C, _FB), lambda f, b, be, used: (be[b], 0, f)),
            pl.BlockSpec(
                (1, 1, _FB), lambda f, b, be, used: (be[b], 0, f)),
            pl.BlockSpec(
                (1, _FB, _C), lambda f, b, be, used: (be[b], f, 0)),
            pl.BlockSpec((1, 1, _C), lambda f, b, be, used: (be[b], 0, 0)),
        ],
        out_specs=pl.BlockSpec(memory_space=pltpu.MemorySpace.HBM),
        scratch_shapes=[
            pltpu.VMEM((_C, _FB), jnp.bfloat16),
            pltpu.VMEM((_FB, _C), jnp.bfloat16),
            pltpu.VMEM((_SPAD, _C), jnp.float32),
            pltpu.SemaphoreType.DMA,
        ],
    )
    return pl.pallas_call(
        _gmm_body,
        grid_spec=grid_spec,
        out_shape=jax.ShapeDtypeStruct((_SPAD, _C), jnp.float32),
        compiler_params=pltpu.CompilerParams(
            dimension_semantics=("arbitrary", "arbitrary")),
        interpret=interpret,
    )(be, used, xg, fc1_W, fc1_b.reshape(_E, 1, _F), fc2_W,
      fc2_b.reshape(_E, 1, _C))


def _dispatch_call(x2d, pos0, pos1):
    mesh = plsc.VectorSubcoreMesh(core_axis_name="c", subcore_axis_name="s")

    @functools.partial(
        pl.kernel, mesh=mesh,
        out_type=jax.ShapeDtypeStruct((_SPAD, _C), jnp.float32),
        scratch_types=[
            pltpu.VMEM((_TPW,), jnp.int32),
            pltpu.VMEM((_TPW,), jnp.int32),
            pltpu.VMEM((_TPW, _C), jnp.float32),
            pltpu.SemaphoreType.DMA,
        ],
    )
    def dispatch(x_hbm, pos0_hbm, pos1_hbm, xg_hbm, idx0_v, idx1_v, rows_v, sem):
        wid = lax.axis_index("s") * _NC + lax.axis_index("c")
        row0 = wid * _TPW
        pltpu.sync_copy(x_hbm.at[pl.ds(row0, _TPW)], rows_v)
        pltpu.sync_copy(pos0_hbm.at[pl.ds(row0, _TPW)], idx0_v)
        pltpu.sync_copy(pos1_hbm.at[pl.ds(row0, _TPW)], idx1_v)
        c0 = pltpu.async_copy(rows_v, xg_hbm.at[idx0_v], sem)
        c1 = pltpu.async_copy(rows_v, xg_hbm.at[idx1_v], sem)
        c0.wait()
        c1.wait()

    return dispatch(x2d, pos0, pos1)


def _combine_call(yg, pos0, pos1, w0e, w1e):
    mesh = plsc.VectorSubcoreMesh(core_axis_name="c", subcore_axis_name="s")
    ch_n = 32

    @functools.partial(
        pl.kernel, mesh=mesh,
        out_type=jax.ShapeDtypeStruct((_T, _C), jnp.float32),
        scratch_types=[
            pltpu.VMEM((ch_n,), jnp.int32),
            pltpu.VMEM((ch_n,), jnp.int32),
            pltpu.VMEM((ch_n, _C), jnp.float32),
            pltpu.VMEM((ch_n, _C), jnp.float32),
            pltpu.VMEM((ch_n, 16), jnp.float32),
            pltpu.VMEM((ch_n, 16), jnp.float32),
            pltpu.SemaphoreType.DMA,
        ],
    )
    def combine(yg_hbm, pos0_hbm, pos1_hbm, w0_hbm, w1_hbm, out_hbm,
                idx0_v, idx1_v, r0_v, r1_v, w0_v, w1_v, sem):
        wid = lax.axis_index("s") * _NC + lax.axis_index("c")
        for ch in range(_TPW // ch_n):
            base = wid * _TPW + ch * ch_n
            pltpu.sync_copy(pos0_hbm.at[pl.ds(base, ch_n)], idx0_v)
            pltpu.sync_copy(pos1_hbm.at[pl.ds(base, ch_n)], idx1_v)
            pltpu.sync_copy(w0_hbm.at[pl.ds(base, ch_n)], w0_v)
            pltpu.sync_copy(w1_hbm.at[pl.ds(base, ch_n)], w1_v)
            g0 = pltpu.async_copy(yg_hbm.at[idx0_v], r0_v, sem)
            g1 = pltpu.async_copy(yg_hbm.at[idx1_v], r1_v, sem)
            g0.wait()
            g1.wait()

            def body(i, carry):
                a = w0_v[i, :]
                b = w1_v[i, :]
                for j in range(_C // 16):
                    sl = pl.ds(j * 16, 16)
                    r0_v[i, sl] = r0_v[i, sl] * a + r1_v[i, sl] * b
                return carry

            lax.fori_loop(0, ch_n, body, 0)
            pltpu.sync_copy(r0_v, out_hbm.at[pl.ds(base, ch_n)])

    return combine(yg, pos0, pos1, w0e, w1e)


def kernel(x, gate_W, gate_b, fc1_W, fc1_b, fc2_W, fc2_b):
    B, T, C = x.shape
    x2d = x.reshape(T, C)
    gwp = jnp.pad(gate_W, ((0, 0), (0, _EPAD - _E)))
    gbp = jnp.pad(gate_b, (0, _EPAD - _E)).reshape(1, _EPAD)
    pos0, pos1, w0e, w1e, be, used = _router_call(x2d, gwp, gbp)
    pos0f = pos0.reshape(_T)
    pos1f = pos1.reshape(_T)
    xg = _dispatch_call(x2d, pos0f, pos1f)
    yg = _gmm_call(be.reshape(_NB), used.reshape(1), xg,
                   fc1_W, fc1_b, fc2_W, fc2_b)
    out = _combine_call(yg, pos0f, pos1f, w0e, w1e)
    return out.reshape(B, T, C)


# speedup vs baseline: 3.2864x; 1.1437x over previous
import functools

import jax
import jax.numpy as jnp
from jax import lax
from jax.experimental import pallas as pl
from jax.experimental.pallas import tpu as pltpu
from jax.experimental.pallas import tpu_sc as plsc

_T = 2048
_C = 1024
_F = 4096
_E = 8
_EPAD = 128
_M = 256
_NB = _T * 2 // _M + _E
_SPAD = _NB * _M
_FB = 2048
_NF = _F // _FB
_NC, _NS = 2, 16
_NW = _NC * _NS
_TPW = _T // _NW


def _gelu_exact(h):
    return 0.5 * h * (1.0 + lax.erf(h * 0.7071067811865476))


def _router_body(x_ref, gw_ref, gb_ref,
                 pos0_ref, pos1_ref, w0_ref, w1_ref, be_ref, used_ref):
    lane = lax.broadcasted_iota(jnp.int32, (_T, _EPAD), 1).astype(jnp.float32)
    valid_row = lax.broadcasted_iota(jnp.int32, (1, _EPAD), 1) < _E
    logits = jnp.dot(x_ref[...], gw_ref[...], preferred_element_type=jnp.float32)
    logits = logits + gb_ref[...]
    neg = -1e30
    l = jnp.where(lane < _E, logits, neg)
    m1 = jnp.max(l, axis=1, keepdims=True)
    i1 = jnp.min(jnp.where(l == m1, lane, 1e9), axis=1, keepdims=True)
    sel1 = lane == i1
    l2 = jnp.where(sel1, neg, l)
    m2 = jnp.max(l2, axis=1, keepdims=True)
    i2 = jnp.min(jnp.where(l2 == m2, lane, 1e9), axis=1, keepdims=True)
    sel2 = lane == i2
    e2 = jnp.exp(m2 - m1)
    w0 = 1.0 / (1.0 + e2)
    w1 = e2 / (1.0 + e2)
    oh = sel1.astype(jnp.float32) + sel2.astype(jnp.float32)
    acc = oh
    k = 1
    while k < _T:
        shifted = jnp.concatenate(
            [jnp.zeros((k, _EPAD), jnp.float32), acc[:_T - k, :]], axis=0)
        acc = acc + shifted
        k *= 2
    ranks = acc - oh
    counts = acc[_T - 1:_T, :]
    ub = jnp.floor((counts + (_M - 1)) * (1.0 / _M))
    pc = ub * _M
    tri = (lax.broadcasted_iota(jnp.int32, (_EPAD, _EPAD), 0) <
           lax.broadcasted_iota(jnp.int32, (_EPAD, _EPAD), 1)).astype(jnp.float32)
    poff = jnp.dot(pc, tri, preferred_element_type=jnp.float32)
    cb = (poff + pc) * (1.0 / _M)
    cb_m = jnp.where(valid_row, cb, 1e9)
    used_f = cb[0:1, _E - 1:_E]
    rowb = lax.broadcasted_iota(jnp.int32, (_NB, _EPAD), 0).astype(jnp.float32)
    be = jnp.sum((rowb >= cb_m).astype(jnp.float32), axis=1, keepdims=True)
    rowcol = lax.broadcasted_iota(jnp.int32, (_NB, 1), 0).astype(jnp.float32)
    be_last = jnp.sum(jnp.where(rowcol == used_f - 1.0, be, 0.0),
                      axis=0, keepdims=True)
    be_fin = jnp.where(rowcol < used_f, be, be_last)
    base = ranks + poff
    pos0 = jnp.sum(jnp.where(sel1, base, 0.0), axis=1, keepdims=True)
    pos1 = jnp.sum(jnp.where(sel2, base, 0.0), axis=1, keepdims=True)
    pos0_ref[...] = pos0.astype(jnp.int32)
    pos1_ref[...] = pos1.astype(jnp.int32)
    w0_ref[...] = jnp.broadcast_to(w0, (_T, 16))
    w1_ref[...] = jnp.broadcast_to(w1, (_T, 16))
    be_ref[...] = be_fin.astype(jnp.int32)
    used_ref[...] = used_f.astype(jnp.int32)


def _router_call(x2d, gwp, gbp, interpret=False):
    return pl.pallas_call(
        _router_body,
        out_shape=[
            jax.ShapeDtypeStruct((_T, 1), jnp.int32),
            jax.ShapeDtypeStruct((_T, 1), jnp.int32),
            jax.ShapeDtypeStruct((_T, 16), jnp.float32),
            jax.ShapeDtypeStruct((_T, 16), jnp.float32),
            jax.ShapeDtypeStruct((_NB, 1), jnp.int32),
            jax.ShapeDtypeStruct((1, 1), jnp.int32),
        ],
        interpret=interpret,
    )(x2d, gwp, gbp)


def _gmm_body(be_s, used_s, xg_ref, w1_ref, b1_ref, w2_ref, b2_ref, out_hbm,
              w1c, w2c, acc, outf, sem):
    f = pl.program_id(0)
    b = pl.program_id(1)
    valid = b < used_s[0]
    off = pl.multiple_of(b * _M, _M)

    e_prev = be_s[jnp.maximum(b - 1, 0)]
    changed = (b == 0) | (be_s[b] != e_prev)

    @pl.when(changed)
    def _cast_w():
        w1c[...] = w1_ref[0].astype(jnp.bfloat16)
        w2c[...] = w2_ref[0].astype(jnp.bfloat16)

    @pl.when(valid)
    def _compute():
        xb = xg_ref[...].astype(jnp.bfloat16)
        h = jnp.dot(xb, w1c[...], preferred_element_type=jnp.float32)
        h = _gelu_exact(h + b1_ref[0])
        upd = jnp.dot(h.astype(jnp.bfloat16), w2c[...],
                      preferred_element_type=jnp.float32)

        @pl.when(f == 0)
        def _():
            acc[pl.ds(off, _M), :] = (
                jnp.broadcast_to(b2_ref[0], (_M, _C)) + upd
            ).astype(jnp.bfloat16)

        @pl.when(f == _NF - 1)
        def _flush():
            outf[...] = acc[pl.ds(off, _M), :].astype(jnp.float32) + upd
            cp = pltpu.make_async_copy(
                outf, out_hbm.at[pl.ds(off, _M)], sem)
            cp.start()
            cp.wait()


def _gmm_call(be, used, xg, fc1_W, fc1_b, fc2_W, fc2_b, interpret=False):
    grid_spec = pltpu.PrefetchScalarGridSpec(
        num_scalar_prefetch=2,
        grid=(_NF, _NB),
        in_specs=[
            pl.BlockSpec(
                (_M, _C),
                lambda f, b, be, used: (
                    jnp.where(b < used[0], b, used[0] - 1), 0)),
            pl.BlockSpec(
                (1, _C, _FB), lambda f, b, be, used: (be[b], 0, f)),
            pl.BlockSpec(
                (1, 1, _FB), lambda f, b, be, used: (be[b], 0, f)),
            pl.BlockSpec(
                (1, _FB, _C), lambda f, b, be, used: (be[b], f, 0)),
            pl.BlockSpec((1, 1, _C), lambda f, b, be, used: (be[b], 0, 0)),
        ],
        out_specs=pl.BlockSpec(memory_space=pltpu.MemorySpace.HBM),
        scratch_shapes=[
            pltpu.VMEM((_C, _FB), jnp.bfloat16),
            pltpu.VMEM((_FB, _C), jnp.bfloat16),
            pltpu.VMEM((_SPAD, _C), jnp.bfloat16),
            pltpu.VMEM((_M, _C), jnp.float32),
            pltpu.SemaphoreType.DMA,
        ],
    )
    return pl.pallas_call(
        _gmm_body,
        grid_spec=grid_spec,
        out_shape=jax.ShapeDtypeStruct((_SPAD, _C), jnp.float32),
        compiler_params=pltpu.CompilerParams(
            dimension_semantics=("arbitrary", "arbitrary")),
        interpret=interpret,
    )(be, used, xg, fc1_W, fc1_b.reshape(_E, 1, _F), fc2_W,
      fc2_b.reshape(_E, 1, _C))


def _dispatch_call(x2d, pos0, pos1):
    mesh = plsc.VectorSubcoreMesh(core_axis_name="c", subcore_axis_name="s")

    @functools.partial(
        pl.kernel, mesh=mesh,
        out_type=jax.ShapeDtypeStruct((_SPAD, _C), jnp.float32),
        scratch_types=[
            pltpu.VMEM((_TPW,), jnp.int32),
            pltpu.VMEM((_TPW,), jnp.int32),
            pltpu.VMEM((_TPW, _C), jnp.float32),
            pltpu.SemaphoreType.DMA,
        ],
    )
    def dispatch(x_hbm, pos0_hbm, pos1_hbm, xg_hbm, idx0_v, idx1_v, rows_v, sem):
        wid = lax.axis_index("s") * _NC + lax.axis_index("c")
        row0 = wid * _TPW
        pltpu.sync_copy(x_hbm.at[pl.ds(row0, _TPW)], rows_v)
        pltpu.sync_copy(pos0_hbm.at[pl.ds(row0, _TPW)], idx0_v)
        pltpu.sync_copy(pos1_hbm.at[pl.ds(row0, _TPW)], idx1_v)
        c0 = pltpu.async_copy(rows_v, xg_hbm.at[idx0_v], sem)
        c1 = pltpu.async_copy(rows_v, xg_hbm.at[idx1_v], sem)
        c0.wait()
        c1.wait()

    return dispatch(x2d, pos0, pos1)


def _combine_call(yg, pos0, pos1, w0e, w1e):
    mesh = plsc.VectorSubcoreMesh(core_axis_name="c", subcore_axis_name="s")
    ch_n = 32

    @functools.partial(
        pl.kernel, mesh=mesh,
        out_type=jax.ShapeDtypeStruct((_T, _C), jnp.float32),
        scratch_types=[
            pltpu.VMEM((ch_n,), jnp.int32),
            pltpu.VMEM((ch_n,), jnp.int32),
            pltpu.VMEM((ch_n, _C), jnp.float32),
            pltpu.VMEM((ch_n, _C), jnp.float32),
            pltpu.VMEM((ch_n, 16), jnp.float32),
            pltpu.VMEM((ch_n, 16), jnp.float32),
            pltpu.SemaphoreType.DMA,
        ],
    )
    def combine(yg_hbm, pos0_hbm, pos1_hbm, w0_hbm, w1_hbm, out_hbm,
                idx0_v, idx1_v, r0_v, r1_v, w0_v, w1_v, sem):
        wid = lax.axis_index("s") * _NC + lax.axis_index("c")
        for ch in range(_TPW // ch_n):
            base = wid * _TPW + ch * ch_n
            pltpu.sync_copy(pos0_hbm.at[pl.ds(base, ch_n)], idx0_v)
            pltpu.sync_copy(pos1_hbm.at[pl.ds(base, ch_n)], idx1_v)
            pltpu.sync_copy(w0_hbm.at[pl.ds(base, ch_n)], w0_v)
            pltpu.sync_copy(w1_hbm.at[pl.ds(base, ch_n)], w1_v)
            g0 = pltpu.async_copy(yg_hbm.at[idx0_v], r0_v, sem)
            g1 = pltpu.async_copy(yg_hbm.at[idx1_v], r1_v, sem)
            g0.wait()
            g1.wait()

            def body(i, carry):
                a = w0_v[i, :]
                b = w1_v[i, :]
                for j in range(_C // 16):
                    sl = pl.ds(j * 16, 16)
                    r0_v[i, sl] = r0_v[i, sl] * a + r1_v[i, sl] * b
                return carry

            lax.fori_loop(0, ch_n, body, 0)
            pltpu.sync_copy(r0_v, out_hbm.at[pl.ds(base, ch_n)])

    return combine(yg, pos0, pos1, w0e, w1e)


def kernel(x, gate_W, gate_b, fc1_W, fc1_b, fc2_W, fc2_b):
    B, T, C = x.shape
    x2d = x.reshape(T, C)
    gwp = jnp.pad(gate_W, ((0, 0), (0, _EPAD - _E)))
    gbp = jnp.pad(gate_b, (0, _EPAD - _E)).reshape(1, _EPAD)
    pos0, pos1, w0e, w1e, be, used = _router_call(x2d, gwp, gbp)
    pos0f = pos0.reshape(_T)
    pos1f = pos1.reshape(_T)
    xg = _dispatch_call(x2d, pos0f, pos1f)
    yg = _gmm_call(be.reshape(_NB), used.reshape(1), xg,
                   fc1_W, fc1_b, fc2_W, fc2_b)
    out = _combine_call(yg, pos0f, pos1f, w0e, w1e)
    return out.reshape(B, T, C)


# speedup vs baseline: 3.3828x; 1.0293x over previous
import functools

import jax
import jax.numpy as jnp
from jax import lax
from jax.experimental import pallas as pl
from jax.experimental.pallas import tpu as pltpu
from jax.experimental.pallas import tpu_sc as plsc

_T = 2048
_C = 1024
_F = 4096
_E = 8
_EPAD = 128
_M = 256
_NB = _T * 2 // _M + _E
_SPAD = _NB * _M
_FB = 2048
_NF = _F // _FB
_NC, _NS = 2, 16
_NW = _NC * _NS
_TPW = _T // _NW


def _gelu_exact(h):
    return 0.5 * h * (1.0 + lax.erf(h * 0.7071067811865476))


def _router_body(x_ref, gw_ref, gb_ref,
                 pos0_ref, pos1_ref, w0_ref, w1_ref, be_ref, used_ref):
    lane = lax.broadcasted_iota(jnp.int32, (_T, _EPAD), 1).astype(jnp.float32)
    valid_row = lax.broadcasted_iota(jnp.int32, (1, _EPAD), 1) < _E
    logits = jnp.dot(x_ref[...], gw_ref[...], preferred_element_type=jnp.float32)
    logits = logits + gb_ref[...]
    neg = -1e30
    l = jnp.where(lane < _E, logits, neg)
    m1 = jnp.max(l, axis=1, keepdims=True)
    i1 = jnp.min(jnp.where(l == m1, lane, 1e9), axis=1, keepdims=True)
    sel1 = lane == i1
    l2 = jnp.where(sel1, neg, l)
    m2 = jnp.max(l2, axis=1, keepdims=True)
    i2 = jnp.min(jnp.where(l2 == m2, lane, 1e9), axis=1, keepdims=True)
    sel2 = lane == i2
    e2 = jnp.exp(m2 - m1)
    w0 = 1.0 / (1.0 + e2)
    w1 = e2 / (1.0 + e2)
    oh = sel1.astype(jnp.float32) + sel2.astype(jnp.float32)
    acc = oh
    k = 1
    while k < _T:
        shifted = jnp.concatenate(
            [jnp.zeros((k, _EPAD), jnp.float32), acc[:_T - k, :]], axis=0)
        acc = acc + shifted
        k *= 2
    ranks = acc - oh
    counts = acc[_T - 1:_T, :]
    ub = jnp.floor((counts + (_M - 1)) * (1.0 / _M))
    pc = ub * _M
    tri = (lax.broadcasted_iota(jnp.int32, (_EPAD, _EPAD), 0) <
           lax.broadcasted_iota(jnp.int32, (_EPAD, _EPAD), 1)).astype(jnp.float32)
    poff = jnp.dot(pc, tri, preferred_element_type=jnp.float32)
    cb = (poff + pc) * (1.0 / _M)
    cb_m = jnp.where(valid_row, cb, 1e9)
    used_f = cb[0:1, _E - 1:_E]
    rowb = lax.broadcasted_iota(jnp.int32, (_NB, _EPAD), 0).astype(jnp.float32)
    be = jnp.sum((rowb >= cb_m).astype(jnp.float32), axis=1, keepdims=True)
    rowcol = lax.broadcasted_iota(jnp.int32, (_NB, 1), 0).astype(jnp.float32)
    be_last = jnp.sum(jnp.where(rowcol == used_f - 1.0, be, 0.0),
                      axis=0, keepdims=True)
    be_fin = jnp.where(rowcol < used_f, be, be_last)
    base = ranks + poff
    pos0 = jnp.sum(jnp.where(sel1, base, 0.0), axis=1, keepdims=True)
    pos1 = jnp.sum(jnp.where(sel2, base, 0.0), axis=1, keepdims=True)
    pos0_ref[...] = pos0.astype(jnp.int32)
    pos1_ref[...] = pos1.astype(jnp.int32)
    w0_ref[...] = jnp.broadcast_to(w0, (_T, 16))
    w1_ref[...] = jnp.broadcast_to(w1, (_T, 16))
    be_ref[...] = be_fin.astype(jnp.int32)
    used_ref[...] = used_f.astype(jnp.int32)


def _router_call(x2d, gwp, gbp, interpret=False):
    return pl.pallas_call(
        _router_body,
        out_shape=[
            jax.ShapeDtypeStruct((_T, 1), jnp.int32),
            jax.ShapeDtypeStruct((_T, 1), jnp.int32),
            jax.ShapeDtypeStruct((_T, 16), jnp.float32),
            jax.ShapeDtypeStruct((_T, 16), jnp.float32),
            jax.ShapeDtypeStruct((_NB, 1), jnp.int32),
            jax.ShapeDtypeStruct((1, 1), jnp.int32),
        ],
        interpret=interpret,
    )(x2d, gwp, gbp)


def _gmm_body(be_s, used_s, xg_ref, w1_ref, b1_ref, w2_ref, b2_ref, out_hbm,
              w1c, w2c, acc, outf, sem):
    f = pl.program_id(0)
    b = pl.program_id(1)
    valid = b < used_s[0]
    off = pl.multiple_of(b * _M, _M)

    e_prev = be_s[jnp.maximum(b - 1, 0)]
    changed = (b == 0) | (be_s[b] != e_prev)

    @pl.when(changed)
    def _cast_w():
        w1c[...] = w1_ref[0].astype(jnp.bfloat16)
        w2c[...] = w2_ref[0].astype(jnp.bfloat16)

    @pl.when(valid)
    def _compute():
        xb = xg_ref[...].astype(jnp.bfloat16)
        h = jnp.dot(xb, w1c[...], preferred_element_type=jnp.float32)
        h = _gelu_exact(h + b1_ref[0])
        upd = jnp.dot(h.astype(jnp.bfloat16), w2c[...],
                      preferred_element_type=jnp.float32)

        @pl.when(f == 0)
        def _():
            acc[pl.ds(off, _M), :] = (
                jnp.broadcast_to(b2_ref[0], (_M, _C)) + upd
            ).astype(jnp.bfloat16)

        @pl.when(f == _NF - 1)
        def _flush():
            par = lax.rem(b, 2)
            outf[par] = acc[pl.ds(off, _M), :].astype(jnp.float32) + upd
            cp = pltpu.make_async_copy(
                outf.at[par], out_hbm.at[pl.ds(off, _M)], sem)
            cp.start()

            @pl.when(b > 0)
            def _wait_prev():
                cp.wait()

            @pl.when(b == used_s[0] - 1)
            def _drain():
                cp.wait()


def _gmm_call(be, used, xg, fc1_W, fc1_b, fc2_W, fc2_b, interpret=False):
    grid_spec = pltpu.PrefetchScalarGridSpec(
        num_scalar_prefetch=2,
        grid=(_NF, _NB),
        in_specs=[
            pl.BlockSpec(
                (_M, _C),
                lambda f, b, be, used: (
                    jnp.where(b < used[0], b, used[0] - 1), 0)),
            pl.BlockSpec(
                (1, _C, _FB), lambda f, b, be, used: (be[b], 0, f)),
            pl.BlockSpec(
                (1, 1, _FB), lambda f, b, be, used: (be[b], 0, f)),
            pl.BlockSpec(
                (1, _FB, _C), lambda f, b, be, used: (be[b], f, 0)),
            pl.BlockSpec((1, 1, _C), lambda f, b, be, used: (be[b], 0, 0)),
        ],
        out_specs=pl.BlockSpec(memory_space=pltpu.MemorySpace.HBM),
        scratch_shapes=[
            pltpu.VMEM((_C, _FB), jnp.bfloat16),
            pltpu.VMEM((_FB, _C), jnp.bfloat16),
            pltpu.VMEM((_SPAD, _C), jnp.bfloat16),
            pltpu.VMEM((2, _M, _C), jnp.float32),
            pltpu.SemaphoreType.DMA,
        ],
    )
    return pl.pallas_call(
        _gmm_body,
        grid_spec=grid_spec,
        out_shape=jax.ShapeDtypeStruct((_SPAD, _C), jnp.float32),
        compiler_params=pltpu.CompilerParams(
            dimension_semantics=("arbitrary", "arbitrary")),
        interpret=interpret,
    )(be, used, xg, fc1_W, fc1_b.reshape(_E, 1, _F), fc2_W,
      fc2_b.reshape(_E, 1, _C))


def _dispatch_call(x2d, pos0, pos1):
    mesh = plsc.VectorSubcoreMesh(core_axis_name="c", subcore_axis_name="s")

    @functools.partial(
        pl.kernel, mesh=mesh,
        out_type=jax.ShapeDtypeStruct((_SPAD, _C), jnp.float32),
        scratch_types=[
            pltpu.VMEM((_TPW,), jnp.int32),
            pltpu.VMEM((_TPW,), jnp.int32),
            pltpu.VMEM((_TPW, _C), jnp.float32),
            pltpu.SemaphoreType.DMA,
        ],
    )
    def dispatch(x_hbm, pos0_hbm, pos1_hbm, xg_hbm, idx0_v, idx1_v, rows_v, sem):
        wid = lax.axis_index("s") * _NC + lax.axis_index("c")
        row0 = wid * _TPW
        pltpu.sync_copy(x_hbm.at[pl.ds(row0, _TPW)], rows_v)
        pltpu.sync_copy(pos0_hbm.at[pl.ds(row0, _TPW)], idx0_v)
        pltpu.sync_copy(pos1_hbm.at[pl.ds(row0, _TPW)], idx1_v)
        c0 = pltpu.async_copy(rows_v, xg_hbm.at[idx0_v], sem)
        c1 = pltpu.async_copy(rows_v, xg_hbm.at[idx1_v], sem)
        c0.wait()
        c1.wait()

    return dispatch(x2d, pos0, pos1)


def _combine_call(yg, pos0, pos1, w0e, w1e):
    mesh = plsc.VectorSubcoreMesh(core_axis_name="c", subcore_axis_name="s")
    ch_n = 32

    @functools.partial(
        pl.kernel, mesh=mesh,
        out_type=jax.ShapeDtypeStruct((_T, _C), jnp.float32),
        scratch_types=[
            pltpu.VMEM((ch_n,), jnp.int32),
            pltpu.VMEM((ch_n,), jnp.int32),
            pltpu.VMEM((ch_n, _C), jnp.float32),
            pltpu.VMEM((ch_n, _C), jnp.float32),
            pltpu.VMEM((ch_n, 16), jnp.float32),
            pltpu.VMEM((ch_n, 16), jnp.float32),
            pltpu.SemaphoreType.DMA,
        ],
    )
    def combine(yg_hbm, pos0_hbm, pos1_hbm, w0_hbm, w1_hbm, out_hbm,
                idx0_v, idx1_v, r0_v, r1_v, w0_v, w1_v, sem):
        wid = lax.axis_index("s") * _NC + lax.axis_index("c")
        for ch in range(_TPW // ch_n):
            base = wid * _TPW + ch * ch_n
            pltpu.sync_copy(pos0_hbm.at[pl.ds(base, ch_n)], idx0_v)
            pltpu.sync_copy(pos1_hbm.at[pl.ds(base, ch_n)], idx1_v)
            pltpu.sync_copy(w0_hbm.at[pl.ds(base, ch_n)], w0_v)
            pltpu.sync_copy(w1_hbm.at[pl.ds(base, ch_n)], w1_v)
            g0 = pltpu.async_copy(yg_hbm.at[idx0_v], r0_v, sem)
            g1 = pltpu.async_copy(yg_hbm.at[idx1_v], r1_v, sem)
            g0.wait()
            g1.wait()

            def body(i, carry):
                a = w0_v[i, :]
                b = w1_v[i, :]
                for j in range(_C // 16):
                    sl = pl.ds(j * 16, 16)
                    r0_v[i, sl] = r0_v[i, sl] * a + r1_v[i, sl] * b
                return carry

            lax.fori_loop(0, ch_n, body, 0)
            pltpu.sync_copy(r0_v, out_hbm.at[pl.ds(base, ch_n)])

    return combine(yg, pos0, pos1, w0e, w1e)


def kernel(x, gate_W, gate_b, fc1_W, fc1_b, fc2_W, fc2_b):
    B, T, C = x.shape
    x2d = x.reshape(T, C)
    gwp = jnp.pad(gate_W, ((0, 0), (0, _EPAD - _E)))
    gbp = jnp.pad(gate_b, (0, _EPAD - _E)).reshape(1, _EPAD)
    pos0, pos1, w0e, w1e, be, used = _router_call(x2d, gwp, gbp)
    pos0f = pos0.reshape(_T)
    pos1f = pos1.reshape(_T)
    xg = _dispatch_call(x2d, pos0f, pos1f)
    yg = _gmm_call(be.reshape(_NB), used.reshape(1), xg,
                   fc1_W, fc1_b, fc2_W, fc2_b)
    out = _combine_call(yg, pos0f, pos1f, w0e, w1e)
    return out.reshape(B, T, C)
